# baseline (speedup 1.0000x reference)
import functools

import jax
import jax.numpy as jnp
from jax import lax
from jax.experimental import pallas as pl
from jax.experimental.pallas import tpu as pltpu
from jax.experimental.pallas import tpu_sc as plsc

N_NODES = 10000
D_HID = 128
D_TGT = 16
N_EDGES = 320000

D_CNT = 16
N_CORES = 2
N_SUBCORES = 16
N_WORKERS = N_CORES * N_SUBCORES
EDGES_PER_W = N_EDGES // N_WORKERS
CHUNK = 80
N_CHUNKS = EDGES_PER_W // CHUNK
N_PHASES = 5
CH_PER_PH = N_CHUNKS // N_PHASES
ROWS_PER_TILE = N_NODES // N_SUBCORES
ZFULL = ROWS_PER_TILE // CHUNK
ZREM = ROWS_PER_TILE - ZFULL * CHUNK

BM = 2000


def _vertex_body(x_ref, w_ref, b_ref, out_ref):
    h = jnp.dot(x_ref[...], w_ref[...], preferred_element_type=jnp.float32)
    out_ref[...] = jnp.maximum(h + b_ref[...], 0.0)


def _vertex_mlp(x, w, b):
    n = x.shape[0]
    return pl.pallas_call(
        _vertex_body,
        grid=(n // BM,),
        in_specs=[
            pl.BlockSpec((BM, D_HID), lambda i: (i, 0)),
            pl.BlockSpec((D_HID, D_HID), lambda i: (0, 0)),
            pl.BlockSpec((1, D_HID), lambda i: (0, 0)),
        ],
        out_specs=pl.BlockSpec((BM, D_HID), lambda i: (i, 0)),
        out_shape=jax.ShapeDtypeStruct((n, D_HID), jnp.float32),
    )(x, w, b.reshape(1, D_HID))


def _sc_body(hp_hbm, ei_hbm, z128_hbm, z16_hbm, out_hbm, cnt_hbm, colv, rowv,
             rows0, rows1, rows2, ones, acc_sh, cnt_sh, sem0, sem1, sem2,
             ses0, ses1, ses2, semc):
    cid = lax.axis_index("c")
    sid = lax.axis_index("s")
    wid = sid * N_CORES + cid
    base_r = sid * ROWS_PER_TILE

    pltpu.sync_copy(ei_hbm.at[0, wid, 0], rowv)
    pltpu.sync_copy(ei_hbm.at[1, wid, 0], colv)
    pltpu.async_copy(hp_hbm.at[colv.at[0]], rows0, sem0)

    pltpu.sync_copy(
        z128_hbm.at[pl.ds(base_r, ROWS_PER_TILE)],
        acc_sh.at[pl.ds(base_r, ROWS_PER_TILE)],
    )
    pltpu.sync_copy(
        z16_hbm.at[pl.ds(base_r, ROWS_PER_TILE)],
        cnt_sh.at[pl.ds(base_r, ROWS_PER_TILE)],
    )

    def orow(i, carry):
        ones[i, :] = jnp.ones((D_CNT,), jnp.float32)
        return carry

    lax.fori_loop(0, CHUNK, orow, 0)
    plsc.subcore_barrier()

    for ph in range(N_PHASES):
        if ph > 0:
            pltpu.sync_copy(ei_hbm.at[0, wid, ph], rowv)
            pltpu.sync_copy(ei_hbm.at[1, wid, ph], colv)
            pltpu.async_copy(hp_hbm.at[colv.at[0]], rows0, sem0)
        pltpu.async_copy(hp_hbm.at[colv.at[1]], rows1, sem1)
        pltpu.async_copy(hp_hbm.at[colv.at[2]], rows2, sem2)
        bufs = ((rows0, sem0, ses0), (rows1, sem1, ses1), (rows2, sem2, ses2))

        def step(i, carry):
            for t in range(3):
                j = 3 * i + t
                buf, sem, ses = bufs[t]
                pltpu.async_copy(ones, cnt_sh.at[rowv.at[j]], semc, add=True)
                pltpu.make_async_copy(hp_hbm.at[colv.at[j]], buf, sem).wait()
                pltpu.async_copy(buf, acc_sh.at[rowv.at[j]], ses, add=True)

                @pl.when(j + 3 < CH_PER_PH)
                def _():
                    pltpu.make_async_copy(
                        buf, acc_sh.at[rowv.at[j]], ses).wait()
                    pltpu.async_copy(hp_hbm.at[colv.at[j + 3]], buf, sem)

            return carry

        lax.fori_loop(0, CH_PER_PH // 3, step, 0)
        jt = CH_PER_PH - 1
        pltpu.async_copy(ones, cnt_sh.at[rowv.at[jt]], semc, add=True)
        pltpu.make_async_copy(hp_hbm.at[colv.at[jt]], rows0, sem0).wait()
        pltpu.async_copy(rows0, acc_sh.at[rowv.at[jt]], ses0, add=True)

        pltpu.make_async_copy(rows0, acc_sh.at[rowv.at[0]], ses0).wait()
        pltpu.make_async_copy(rows1, acc_sh.at[rowv.at[0]], ses1).wait()
        pltpu.make_async_copy(rows2, acc_sh.at[rowv.at[0]], ses2).wait()

        def drain(i, carry):
            pltpu.make_async_copy(ones, cnt_sh.at[rowv.at[0]], semc).wait()
            return carry

        lax.fori_loop(0, CH_PER_PH, drain, 0)
    plsc.subcore_barrier()

    pltpu.sync_copy(
        acc_sh.at[pl.ds(base_r, ROWS_PER_TILE)],
        out_hbm.at[cid, pl.ds(base_r, ROWS_PER_TILE)],
    )
    pltpu.sync_copy(
        cnt_sh.at[pl.ds(base_r, ROWS_PER_TILE)],
        cnt_hbm.at[cid, pl.ds(base_r, ROWS_PER_TILE)],
    )


_sc_aggregate = functools.partial(
    pl.kernel,
    out_type=[
        jax.ShapeDtypeStruct((N_CORES, N_NODES, D_HID), jnp.float32),
        jax.ShapeDtypeStruct((N_CORES, N_NODES, D_CNT), jnp.float32),
    ],
    mesh=plsc.VectorSubcoreMesh(core_axis_name="c", subcore_axis_name="s"),
    compiler_params=pltpu.CompilerParams(use_tc_tiling_on_sc=False),
    scratch_types=[
        pltpu.VMEM((CH_PER_PH, CHUNK), jnp.int32),
        pltpu.VMEM((CH_PER_PH, CHUNK), jnp.int32),
        pltpu.VMEM((CHUNK, D_HID), jnp.float32),
        pltpu.VMEM((CHUNK, D_HID), jnp.float32),
        pltpu.VMEM((CHUNK, D_HID), jnp.float32),
        pltpu.VMEM((CHUNK, D_CNT), jnp.float32),
        pltpu.VMEM_SHARED((N_NODES, D_HID), jnp.float32),
        pltpu.VMEM_SHARED((N_NODES, D_CNT), jnp.float32),
        pltpu.SemaphoreType.DMA,
        pltpu.SemaphoreType.DMA,
        pltpu.SemaphoreType.DMA,
        pltpu.SemaphoreType.DMA,
        pltpu.SemaphoreType.DMA,
        pltpu.SemaphoreType.DMA,
        pltpu.SemaphoreType.DMA,
    ],
)(_sc_body)


def _head_body(p_ref, cnt_ref, w1_ref, b1_ref, w2_ref, b2_ref, out_ref):
    s = p_ref[0] + p_ref[1]
    q = cnt_ref[0] + cnt_ref[1]
    c = jnp.max(q, axis=1, keepdims=True)
    c = jnp.where(c == 0.0, 1.0, c)
    agg = s / c
    g = jnp.dot(agg, w1_ref[...], preferred_element_type=jnp.float32)
    g = jnp.maximum(g + b1_ref[...], 0.0)
    o = jnp.dot(g, w2_ref[...], preferred_element_type=jnp.float32)
    out_ref[...] = o + b2_ref[...]


def _head(p, cnt, w1, b1, w2, b2):
    return pl.pallas_call(
        _head_body,
        grid=(N_NODES // BM,),
        in_specs=[
            pl.BlockSpec((N_CORES, BM, D_HID), lambda i: (0, i, 0)),
            pl.BlockSpec((N_CORES, BM, D_CNT), lambda i: (0, i, 0)),
            pl.BlockSpec((D_HID, D_HID), lambda i: (0, 0)),
            pl.BlockSpec((1, D_HID), lambda i: (0, 0)),
            pl.BlockSpec((D_HID, D_TGT), lambda i: (0, 0)),
            pl.BlockSpec((1, D_TGT), lambda i: (0, 0)),
        ],
        out_specs=pl.BlockSpec((BM, D_TGT), lambda i: (i, 0)),
        out_shape=jax.ShapeDtypeStruct((N_NODES, D_TGT), jnp.float32),
    )(p, cnt, w1, b1.reshape(1, D_HID), w2, b2.reshape(1, D_TGT))


@jax.jit
def kernel(x, edge_index, W_vertex, b_vertex, W_g1, b_g1, W_g2, b_g2):
    ei = edge_index.astype(jnp.int32).reshape(
        2, N_WORKERS, N_PHASES, CH_PER_PH, CHUNK)
    hp = _vertex_mlp(x, W_vertex, b_vertex)
    z128 = jnp.zeros((N_NODES, D_HID), jnp.float32)
    z16 = jnp.zeros((N_NODES, D_CNT), jnp.float32)
    p, cnt = _sc_aggregate(hp, ei, z128, z16)
    return _head(p, cnt, W_g1, b_g1, W_g2, b_g2)


# speedup vs baseline: 1.0402x; 1.0402x over previous
import functools

import jax
import jax.numpy as jnp
from jax import lax
from jax.experimental import pallas as pl
from jax.experimental.pallas import tpu as pltpu
from jax.experimental.pallas import tpu_sc as plsc

N_NODES = 10000
D_HID = 128
D_TGT = 16
N_EDGES = 320000

D_CNT = 16
N_CORES = 2
N_SUBCORES = 16
N_WORKERS = N_CORES * N_SUBCORES
EDGES_PER_W = N_EDGES // N_WORKERS
CHUNK = 80
N_CHUNKS = EDGES_PER_W // CHUNK
N_PHASES = 5
CH_PER_PH = N_CHUNKS // N_PHASES
ROWS_PER_TILE = N_NODES // N_SUBCORES
ZFULL = ROWS_PER_TILE // CHUNK
ZREM = ROWS_PER_TILE - ZFULL * CHUNK

BM = 2000


def _vertex_body(x_ref, w_ref, b_ref, out_ref):
    h = jnp.dot(x_ref[...], w_ref[...], preferred_element_type=jnp.float32)
    out_ref[...] = jnp.maximum(h + b_ref[...], 0.0)


def _vertex_mlp(x, w, b):
    n = x.shape[0]
    return pl.pallas_call(
        _vertex_body,
        grid=(n // BM,),
        in_specs=[
            pl.BlockSpec((BM, D_HID), lambda i: (i, 0)),
            pl.BlockSpec((D_HID, D_HID), lambda i: (0, 0)),
            pl.BlockSpec((1, D_HID), lambda i: (0, 0)),
        ],
        out_specs=pl.BlockSpec((BM, D_HID), lambda i: (i, 0)),
        out_shape=jax.ShapeDtypeStruct((n, D_HID), jnp.float32),
    )(x, w, b.reshape(1, D_HID))


def _sc_body(hp_hbm, ei_hbm, out_hbm, cnt_hbm, colv, rowv,
             rows0, rows1, rows2, ones, acc_sh, cnt_sh, sem0, sem1, sem2,
             ses0, ses1, ses2, semc):
    cid = lax.axis_index("c")
    sid = lax.axis_index("s")
    wid = sid * N_CORES + cid
    base_r = sid * ROWS_PER_TILE

    pltpu.sync_copy(ei_hbm.at[0, wid, 0], rowv)
    pltpu.sync_copy(ei_hbm.at[1, wid, 0], colv)
    pltpu.async_copy(hp_hbm.at[colv.at[0]], rows0, sem0)

    def zrow(i, carry):
        for j in range(D_HID // 16):
            rows1[i, pl.ds(j * 16, 16)] = jnp.zeros((16,), jnp.float32)
        ones[i, :] = jnp.zeros((D_CNT,), jnp.float32)
        return carry

    lax.fori_loop(0, CHUNK, zrow, 0)
    for k in range(ZFULL):
        pltpu.sync_copy(rows1, acc_sh.at[pl.ds(base_r + k * CHUNK, CHUNK)])
        pltpu.sync_copy(ones, cnt_sh.at[pl.ds(base_r + k * CHUNK, CHUNK)])
    pltpu.sync_copy(
        rows1.at[pl.ds(0, ZREM)],
        acc_sh.at[pl.ds(base_r + ZFULL * CHUNK, ZREM)],
    )
    pltpu.sync_copy(
        ones.at[pl.ds(0, ZREM)],
        cnt_sh.at[pl.ds(base_r + ZFULL * CHUNK, ZREM)],
    )

    def orow(i, carry):
        ones[i, :] = jnp.ones((D_CNT,), jnp.float32)
        return carry

    lax.fori_loop(0, CHUNK, orow, 0)
    plsc.subcore_barrier()

    for ph in range(N_PHASES):
        if ph > 0:
            pltpu.sync_copy(ei_hbm.at[0, wid, ph], rowv)
            pltpu.sync_copy(ei_hbm.at[1, wid, ph], colv)
            pltpu.async_copy(hp_hbm.at[colv.at[0]], rows0, sem0)
        pltpu.async_copy(hp_hbm.at[colv.at[1]], rows1, sem1)
        pltpu.async_copy(hp_hbm.at[colv.at[2]], rows2, sem2)
        bufs = ((rows0, sem0, ses0), (rows1, sem1, ses1), (rows2, sem2, ses2))

        def step(i, carry):
            for t in range(3):
                j = 3 * i + t
                buf, sem, ses = bufs[t]
                pltpu.async_copy(ones, cnt_sh.at[rowv.at[j]], semc, add=True)
                pltpu.make_async_copy(hp_hbm.at[colv.at[j]], buf, sem).wait()
                pltpu.async_copy(buf, acc_sh.at[rowv.at[j]], ses, add=True)

                @pl.when(j + 3 < CH_PER_PH)
                def _():
                    pltpu.make_async_copy(
                        buf, acc_sh.at[rowv.at[j]], ses).wait()
                    pltpu.async_copy(hp_hbm.at[colv.at[j + 3]], buf, sem)

            return carry

        lax.fori_loop(0, CH_PER_PH // 3, step, 0)
        jt = CH_PER_PH - 1
        pltpu.async_copy(ones, cnt_sh.at[rowv.at[jt]], semc, add=True)
        pltpu.make_async_copy(hp_hbm.at[colv.at[jt]], rows0, sem0).wait()
        pltpu.async_copy(rows0, acc_sh.at[rowv.at[jt]], ses0, add=True)

        pltpu.make_async_copy(rows0, acc_sh.at[rowv.at[0]], ses0).wait()
        pltpu.make_async_copy(rows1, acc_sh.at[rowv.at[0]], ses1).wait()
        pltpu.make_async_copy(rows2, acc_sh.at[rowv.at[0]], ses2).wait()

        def drain(i, carry):
            pltpu.make_async_copy(ones, cnt_sh.at[rowv.at[0]], semc).wait()
            return carry

        lax.fori_loop(0, CH_PER_PH, drain, 0)
    plsc.subcore_barrier()

    pltpu.sync_copy(
        acc_sh.at[pl.ds(base_r, ROWS_PER_TILE)],
        out_hbm.at[cid, pl.ds(base_r, ROWS_PER_TILE)],
    )
    pltpu.sync_copy(
        cnt_sh.at[pl.ds(base_r, ROWS_PER_TILE)],
        cnt_hbm.at[cid, pl.ds(base_r, ROWS_PER_TILE)],
    )


_sc_aggregate = functools.partial(
    pl.kernel,
    out_type=[
        jax.ShapeDtypeStruct((N_CORES, N_NODES, D_HID), jnp.float32),
        jax.ShapeDtypeStruct((N_CORES, N_NODES, D_CNT), jnp.float32),
    ],
    mesh=plsc.VectorSubcoreMesh(core_axis_name="c", subcore_axis_name="s"),
    compiler_params=pltpu.CompilerParams(use_tc_tiling_on_sc=False),
    scratch_types=[
        pltpu.VMEM((CH_PER_PH, CHUNK), jnp.int32),
        pltpu.VMEM((CH_PER_PH, CHUNK), jnp.int32),
        pltpu.VMEM((CHUNK, D_HID), jnp.float32),
        pltpu.VMEM((CHUNK, D_HID), jnp.float32),
        pltpu.VMEM((CHUNK, D_HID), jnp.float32),
        pltpu.VMEM((CHUNK, D_CNT), jnp.float32),
        pltpu.VMEM_SHARED((N_NODES, D_HID), jnp.float32),
        pltpu.VMEM_SHARED((N_NODES, D_CNT), jnp.float32),
        pltpu.SemaphoreType.DMA,
        pltpu.SemaphoreType.DMA,
        pltpu.SemaphoreType.DMA,
        pltpu.SemaphoreType.DMA,
        pltpu.SemaphoreType.DMA,
        pltpu.SemaphoreType.DMA,
        pltpu.SemaphoreType.DMA,
    ],
)(_sc_body)


def _head_body(p_ref, cnt_ref, w1_ref, b1_ref, w2_ref, b2_ref, out_ref):
    s = p_ref[0] + p_ref[1]
    q = cnt_ref[0] + cnt_ref[1]
    c = jnp.max(q, axis=1, keepdims=True)
    c = jnp.where(c == 0.0, 1.0, c)
    agg = s / c
    g = jnp.dot(agg, w1_ref[...], preferred_element_type=jnp.float32)
    g = jnp.maximum(g + b1_ref[...], 0.0)
    o = jnp.dot(g, w2_ref[...], preferred_element_type=jnp.float32)
    out_ref[...] = o + b2_ref[...]


def _head(p, cnt, w1, b1, w2, b2):
    return pl.pallas_call(
        _head_body,
        grid=(N_NODES // BM,),
        in_specs=[
            pl.BlockSpec((N_CORES, BM, D_HID), lambda i: (0, i, 0)),
            pl.BlockSpec((N_CORES, BM, D_CNT), lambda i: (0, i, 0)),
            pl.BlockSpec((D_HID, D_HID), lambda i: (0, 0)),
            pl.BlockSpec((1, D_HID), lambda i: (0, 0)),
            pl.BlockSpec((D_HID, D_TGT), lambda i: (0, 0)),
            pl.BlockSpec((1, D_TGT), lambda i: (0, 0)),
        ],
        out_specs=pl.BlockSpec((BM, D_TGT), lambda i: (i, 0)),
        out_shape=jax.ShapeDtypeStruct((N_NODES, D_TGT), jnp.float32),
    )(p, cnt, w1, b1.reshape(1, D_HID), w2, b2.reshape(1, D_TGT))


@jax.jit
def kernel(x, edge_index, W_vertex, b_vertex, W_g1, b_g1, W_g2, b_g2):
    ei = edge_index.astype(jnp.int32).reshape(
        2, N_WORKERS, N_PHASES, CH_PER_PH, CHUNK)
    hp = _vertex_mlp(x, W_vertex, b_vertex)
    p, cnt = _sc_aggregate(hp, ei)
    return _head(p, cnt, W_g1, b_g1, W_g2, b_g2)
